# baseline (speedup 1.0000x reference)
import jax
import jax.numpy as jnp
from jax.experimental import pallas as pl
from jax.experimental.pallas import tpu as pltpu

_B = 8


def _pos_kernel(col_ref, row_ref, out_ref, patt_ref, sems):
    col = col_ref[...]
    row = row_ref[...]
    h, w = row.shape[0], col.shape[0]
    d = col.shape[1]
    colpat = jnp.broadcast_to(col[None], (h, w, d)).reshape(h * w, d)
    rowpat = jnp.broadcast_to(row[:, None, :], (h, w, d)).reshape(h * w, d)
    patt_ref[...] = jnp.concatenate([colpat, rowpat], axis=1)
    cp0 = pltpu.make_async_copy(patt_ref, out_ref.at[0], sems.at[0])
    cp0.start()
    cp0.wait()
    copies = [
        pltpu.make_async_copy(out_ref.at[0], out_ref.at[b], sems.at[b])
        for b in range(1, _B)
    ]
    for c in copies:
        c.start()
    for c in copies:
        c.wait()


def kernel(x, row_embed, col_embed):
    b = x.shape[0]
    h, w = x.shape[-2], x.shape[-1]
    d = col_embed.shape[-1]
    col = col_embed[:w]
    row = row_embed[:h]
    out = pl.pallas_call(
        _pos_kernel,
        in_specs=[
            pl.BlockSpec(memory_space=pltpu.VMEM),
            pl.BlockSpec(memory_space=pltpu.VMEM),
        ],
        out_specs=pl.BlockSpec(memory_space=pl.MemorySpace.ANY),
        out_shape=jax.ShapeDtypeStruct((b, h * w, 2 * d), jnp.float32),
        scratch_shapes=[
            pltpu.VMEM((h * w, 2 * d), jnp.float32),
            pltpu.SemaphoreType.DMA((b,)),
        ],
    )(col, row)
    return out.reshape(b, h, w, 2 * d).transpose(0, 3, 1, 2)


# speedup vs baseline: 46.3458x; 46.3458x over previous
import jax
import jax.numpy as jnp
from jax.experimental import pallas as pl
from jax.experimental.pallas import tpu as pltpu


def _pos_kernel(col_ref, row_ref, out_ref, patt_ref, sems):
    col = col_ref[...]
    row = row_ref[...]
    h, w = row.shape[0], col.shape[0]
    d = col.shape[1]
    colpat = jnp.broadcast_to(col[None], (h, w, d)).reshape(h * w, d)
    rowpat = jnp.broadcast_to(row[:, None, :], (h, w, d)).reshape(h * w, d)
    patt_ref[...] = jnp.concatenate([colpat, rowpat], axis=1)
    copies = [
        pltpu.make_async_copy(patt_ref, out_ref.at[b], sems.at[b])
        for b in range(out_ref.shape[0])
    ]
    for c in copies:
        c.start()
    for c in copies:
        c.wait()


def kernel(x, row_embed, col_embed):
    b = x.shape[0]
    h, w = x.shape[-2], x.shape[-1]
    d = col_embed.shape[-1]
    col = col_embed[:w]
    row = row_embed[:h]
    out = pl.pallas_call(
        _pos_kernel,
        in_specs=[
            pl.BlockSpec(memory_space=pltpu.VMEM),
            pl.BlockSpec(memory_space=pltpu.VMEM),
        ],
        out_specs=pl.BlockSpec(memory_space=pl.MemorySpace.ANY),
        out_shape=jax.ShapeDtypeStruct((b, h * w, 2 * d), jnp.float32),
        scratch_shapes=[
            pltpu.VMEM((h * w, 2 * d), jnp.float32),
            pltpu.SemaphoreType.DMA((b,)),
        ],
    )(col, row)
    return out.reshape(b, h, w, 2 * d).transpose(0, 3, 1, 2)


# speedup vs baseline: 46.5727x; 1.0049x over previous
import jax
import jax.numpy as jnp
from jax.experimental import pallas as pl
from jax.experimental.pallas import tpu as pltpu


def _pos_kernel(col_ref, row_ref, out_ref, patt_ref, sems):
    col = col_ref[...]
    row = row_ref[...]
    h, w = row.shape[0], col.shape[0]
    d = col.shape[1]
    nb = out_ref.shape[0]
    hh = h // 2
    half = hh * w

    def build(rows):
        colpat = jnp.broadcast_to(col[None], (hh, w, d)).reshape(half, d)
        rowpat = jnp.broadcast_to(rows[:, None, :], (hh, w, d)).reshape(half, d)
        return jnp.concatenate([colpat, rowpat], axis=1)

    patt_ref[pl.ds(0, half), :] = build(row[:hh])
    wave1 = [
        pltpu.make_async_copy(
            patt_ref.at[pl.ds(0, half)],
            out_ref.at[b, pl.ds(0, half)],
            sems.at[b],
        )
        for b in range(nb)
    ]
    for c in wave1:
        c.start()
    patt_ref[pl.ds(half, half), :] = build(row[hh:])
    wave2 = [
        pltpu.make_async_copy(
            patt_ref.at[pl.ds(half, half)],
            out_ref.at[b, pl.ds(half, half)],
            sems.at[nb + b],
        )
        for b in range(nb)
    ]
    for c in wave2:
        c.start()
    for c in wave1 + wave2:
        c.wait()


def kernel(x, row_embed, col_embed):
    b = x.shape[0]
    h, w = x.shape[-2], x.shape[-1]
    d = col_embed.shape[-1]
    col = col_embed[:w]
    row = row_embed[:h]
    out = pl.pallas_call(
        _pos_kernel,
        in_specs=[
            pl.BlockSpec(memory_space=pltpu.VMEM),
            pl.BlockSpec(memory_space=pltpu.VMEM),
        ],
        out_specs=pl.BlockSpec(memory_space=pl.MemorySpace.ANY),
        out_shape=jax.ShapeDtypeStruct((b, h * w, 2 * d), jnp.float32),
        scratch_shapes=[
            pltpu.VMEM((h * w, 2 * d), jnp.float32),
            pltpu.SemaphoreType.DMA((2 * b,)),
        ],
    )(col, row)
    return out.reshape(b, h, w, 2 * d).transpose(0, 3, 1, 2)
